# baseline (speedup 1.0000x reference)
import functools

import jax
import jax.numpy as jnp
from jax import lax
from jax.experimental import pallas as pl
from jax.experimental.pallas import tpu as pltpu
from jax.experimental.pallas import tpu_sc as plsc

B = 4
S = 2048
J = 4096
D = 128
DP = D // 2

NC = 2
NS = 16
GPC = B // NC
TPG = NS // GPC
JPT = J // TPG
CHUNK = 128
NCHUNK = JPT // CHUNK
LANES = 16


def _pack_halves(x):
    lo = x[..., :DP].astype(jnp.bfloat16).astype(jnp.float32)
    hi = x[..., DP:].astype(jnp.bfloat16).astype(jnp.float32)
    lo_u = lax.shift_right_logical(lax.bitcast_convert_type(lo, jnp.uint32), jnp.uint32(16))
    hi_u = lax.bitcast_convert_type(hi, jnp.uint32) & jnp.uint32(0xFFFF0000)
    return lax.bitcast_convert_type(lo_u | hi_u, jnp.float32)


def _proj_body(x_ref, w0_ref, b0_ref, w1_ref, b1_ref, a_ref, c_ref, aux_ref):
    x = x_ref[0]
    wd = w0_ref[:D, :]
    wa = w0_ref[D:, :]
    a = jnp.dot(x, wd, preferred_element_type=jnp.float32) + b0_ref[...]
    c = jnp.dot(x, wa, preferred_element_type=jnp.float32)
    a_ref[...] = _pack_halves(a)
    c_ref[...] = _pack_halves(c)

    @pl.when(pl.program_id(0) == 0)
    def _():
        aux_ref[0:1, :] = _pack_halves(w1_ref[...])
        aux_ref[1:2, :] = jnp.broadcast_to(b1_ref[...], (1, DP))


def _project(x3, w0, b0row, w1row, b1row):
    nbpg = 4
    rows = S // nbpg
    nblk = B * nbpg
    return pl.pallas_call(
        _proj_body,
        grid=(nblk,),
        in_specs=[
            pl.BlockSpec((1, rows, D), lambda i: (i // nbpg, i % nbpg, 0)),
            pl.BlockSpec((2 * D, D), lambda i: (0, 0)),
            pl.BlockSpec((1, D), lambda i: (0, 0)),
            pl.BlockSpec((1, D), lambda i: (0, 0)),
            pl.BlockSpec((1, 1), lambda i: (0, 0)),
        ],
        out_specs=[
            pl.BlockSpec((rows, DP), lambda i: (i, 0)),
            pl.BlockSpec((rows, DP), lambda i: (i, 0)),
            pl.BlockSpec((2, DP), lambda i: (0, 0)),
        ],
        out_shape=[
            jax.ShapeDtypeStruct((B * S, DP), jnp.float32),
            jax.ShapeDtypeStruct((B * S, DP), jnp.float32),
            jax.ShapeDtypeStruct((2, DP), jnp.float32),
        ],
    )(x3, w0, b0row, w1row, b1row)


def _unpack16(v):
    u = lax.bitcast_convert_type(v, jnp.uint32)
    lo = lax.bitcast_convert_type(lax.shift_left(u, jnp.uint32(16)), jnp.float32)
    hi = lax.bitcast_convert_type(u & jnp.uint32(0xFFFF0000), jnp.float32)
    return lo, hi


def _sc_body(a_hbm, c_hbm, dons_hbm, accs_hbm, aux_hbm,
             probs_out, sites_out,
             don_v, acc_v, gd_v, ga_v, a_rows, c_rows, probs_v,
             site_v, aux_v, idx2_v, shared_v, stage_a, stage_c,
             sa0, sa1, sc0, sc1, ss0, ss1):
    c = lax.axis_index("c")
    s = lax.axis_index("s")
    g_local = s // TPG
    g = c * GPC + g_local
    jbase = (s % TPG) * JPT
    flat = g * J + jbase

    srows = GPC * S // NS
    sbase = s * srows
    hbase = c * GPC * S + sbase
    cps0 = pltpu.async_copy(
        a_hbm.at[pl.ds(hbase, srows)], stage_a.at[pl.ds(sbase, srows)], ss0)
    cps1 = pltpu.async_copy(
        c_hbm.at[pl.ds(hbase, srows)], stage_c.at[pl.ds(sbase, srows)], ss1)

    pltpu.sync_copy(dons_hbm.at[g].at[pl.ds(jbase, JPT)], don_v)
    pltpu.sync_copy(accs_hbm.at[g].at[pl.ds(jbase, JPT)], acc_v)
    pltpu.sync_copy(aux_hbm, aux_v)

    zero16 = jnp.zeros((LANES,), jnp.float32)

    def zbody(i, _):
        site_v[i // 8, pl.ds((i % 8) * LANES, LANES)] = zero16
        return 0

    lax.fori_loop(0, S // LANES, zbody, 0)

    @pl.when(s == 0)
    def _():
        pltpu.sync_copy(site_v, shared_v.at[pl.ds(0, NS)])
        pltpu.sync_copy(site_v, shared_v.at[pl.ds(NS, NS)])

    goff = g_local * S
    for i in range(JPT // LANES):
        dvec = don_v[pl.ds(i * LANES, LANES)] + goff
        avec = acc_v[pl.ds(i * LANES, LANES)] + goff
        gd_v[i // (CHUNK // LANES), pl.ds((i % (CHUNK // LANES)) * LANES, LANES)] = dvec
        ga_v[i // (CHUNK // LANES), pl.ds((i % (CHUNK // LANES)) * LANES, LANES)] = avec

    cps0.wait()
    cps1.wait()
    plsc.subcore_barrier()

    ngrp = CHUNK // LANES
    jvecs = [gi * LANES + lax.iota(jnp.int32, LANES) for gi in range(ngrp)]
    sems_a = [sa0, sa1]
    sems_c = [sc0, sc1]

    def start(ch):
        buf = ch % 2
        cpa = pltpu.async_copy(stage_a.at[gd_v.at[ch]], a_rows.at[buf], sems_a[buf])
        cpc = pltpu.async_copy(stage_c.at[ga_v.at[ch]], c_rows.at[buf], sems_c[buf])
        return cpa, cpc

    def compute(ch, cpa, cpc):
        buf = ch % 2
        cpa.wait()
        cpc.wait()
        ar = a_rows.at[buf]
        cr = c_rows.at[buf]

        def k_body(kk, accs):
            kvd = (jnp.zeros((LANES,), jnp.int32) + kk
                   + lax.iota(jnp.int32, LANES)) & (DP - 1)
            w_lo, w_hi = _unpack16(plsc.load_gather(aux_v, [kvd - kvd, kvd]))
            out = []
            for gi in range(ngrp):
                a_lo, a_hi = _unpack16(plsc.load_gather(ar, [jvecs[gi], kvd]))
                c_lo, c_hi = _unpack16(plsc.load_gather(cr, [jvecs[gi], kvd]))
                h_lo = jnp.maximum(a_lo + c_lo, 0.0)
                h_hi = jnp.maximum(a_hi + c_hi, 0.0)
                out.append(accs[gi] + h_lo * w_lo + h_hi * w_hi)
            return tuple(out)

        accs = lax.fori_loop(
            0, DP, k_body,
            tuple(jnp.zeros((LANES,), jnp.float32) for _ in range(ngrp)))
        for gi in range(ngrp):
            probs_v[pl.ds(ch * CHUNK + gi * LANES, LANES)] = accs[gi]

    pend = start(0)
    for ch in range(NCHUNK):
        nxt = start(ch + 1) if ch + 1 < NCHUNK else None
        compute(ch, *pend)
        pend = nxt

    ones = jnp.zeros((LANES,), jnp.int32) + 1
    b1v = plsc.load_gather(aux_v, [ones, ones - 1])

    def sig_body(i, _):
        x = probs_v[pl.ds(i * LANES, LANES)] + b1v
        probs_v[pl.ds(i * LANES, LANES)] = 1.0 / (1.0 + jnp.exp(-x))
        return 0

    lax.fori_loop(0, JPT // LANES, sig_body, 0)
    pltpu.sync_copy(probs_v, probs_out.at[g].at[pl.ds(jbase, JPT)])

    def scat_body(i, _):
        pv = probs_v[pl.ds(i * LANES, LANES)]
        dvec = don_v[pl.ds(i * LANES, LANES)]
        avec = acc_v[pl.ds(i * LANES, LANES)]
        plsc.addupdate_scatter(
            site_v, [lax.shift_right_logical(dvec, 7), dvec & 127], pv)
        plsc.addupdate_scatter(
            site_v, [lax.shift_right_logical(avec, 7), avec & 127], pv)
        return 0

    lax.fori_loop(0, JPT // LANES, scat_body, 0)

    idx2_v[0, pl.ds(0, LANES)] = g_local * NS + lax.iota(jnp.int32, LANES)
    pltpu.sync_copy(site_v, shared_v.at[idx2_v.at[0]], add=True)
    plsc.subcore_barrier()

    pltpu.sync_copy(shared_v.at[s],
                    sites_out.at[c * GPC].at[pl.ds(s * D, D)])
    pltpu.sync_copy(shared_v.at[s + NS],
                    sites_out.at[c * GPC + 1].at[pl.ds(s * D, D)])


def _sc_junctions(a_tab, c_tab, dons_flat, accs_flat, aux):
    mesh = plsc.VectorSubcoreMesh(core_axis_name="c", subcore_axis_name="s")
    f = functools.partial(
        pl.kernel,
        out_type=[
            jax.ShapeDtypeStruct((B, J), jnp.float32),
            jax.ShapeDtypeStruct((B, S), jnp.float32),
        ],
        mesh=mesh,
        compiler_params=pltpu.CompilerParams(needs_layout_passes=False, use_tc_tiling_on_sc=False),
        scratch_types=[
            pltpu.VMEM((JPT,), jnp.int32),
            pltpu.VMEM((JPT,), jnp.int32),
            pltpu.VMEM((NCHUNK, CHUNK), jnp.int32),
            pltpu.VMEM((NCHUNK, CHUNK), jnp.int32),
            pltpu.VMEM((2, CHUNK, DP), jnp.float32),
            pltpu.VMEM((2, CHUNK, DP), jnp.float32),
            pltpu.VMEM((JPT,), jnp.float32),
            pltpu.VMEM((NS, D), jnp.float32),
            pltpu.VMEM((2, DP), jnp.float32),
            pltpu.VMEM((1, LANES), jnp.int32),
            pltpu.VMEM_SHARED((GPC * NS, D), jnp.float32),
            pltpu.VMEM_SHARED((GPC * S, DP), jnp.float32),
            pltpu.VMEM_SHARED((GPC * S, DP), jnp.float32),
            pltpu.SemaphoreType.DMA,
            pltpu.SemaphoreType.DMA,
            pltpu.SemaphoreType.DMA,
            pltpu.SemaphoreType.DMA,
            pltpu.SemaphoreType.DMA,
            pltpu.SemaphoreType.DMA,
        ],
    )(_sc_body)
    return f(a_tab, c_tab, dons_flat, accs_flat, aux)


def kernel(splice_site_reps, gene_start_rep, gene_end_rep, W0, b0, W1, b1,
           junction_dons, junction_accs):
    b0row = b0.reshape(1, D)
    w1row = W1.reshape(1, D)
    b1row = b1.reshape(1, 1)
    a_tab, c_tab, aux = _project(splice_site_reps, W0, b0row, w1row, b1row)

    dons = junction_dons.astype(jnp.int32)
    accs = junction_accs.astype(jnp.int32)

    probs, sites = _sc_junctions(a_tab, c_tab, dons, accs, aux)
    return probs, sites


# speedup vs baseline: 1.0520x; 1.0520x over previous
import functools

import jax
import jax.numpy as jnp
from jax import lax
from jax.experimental import pallas as pl
from jax.experimental.pallas import tpu as pltpu
from jax.experimental.pallas import tpu_sc as plsc

B = 4
S = 2048
J = 4096
D = 128
DP = D // 2

NC = 2
NS = 16
GPC = B // NC
TPG = NS // GPC
JPT = J // TPG
CHUNK = 128
NCHUNK = JPT // CHUNK
LANES = 16


def _pack_halves(x):
    lo = x[..., :DP].astype(jnp.bfloat16).astype(jnp.float32)
    hi = x[..., DP:].astype(jnp.bfloat16).astype(jnp.float32)
    lo_u = lax.shift_right_logical(lax.bitcast_convert_type(lo, jnp.uint32), jnp.uint32(16))
    hi_u = lax.bitcast_convert_type(hi, jnp.uint32) & jnp.uint32(0xFFFF0000)
    return lax.bitcast_convert_type(lo_u | hi_u, jnp.float32)


def _proj_body(x_ref, w0_ref, b0_ref, w1_ref, b1_ref, a_ref, c_ref, aux_ref):
    x = x_ref[0]
    wd = w0_ref[:D, :]
    wa = w0_ref[D:, :]
    a = jnp.dot(x, wd, preferred_element_type=jnp.float32) + b0_ref[...]
    c = jnp.dot(x, wa, preferred_element_type=jnp.float32)
    a_ref[...] = a
    c_ref[...] = c

    @pl.when(pl.program_id(0) == 0)
    def _():
        aux_ref[0:1, :] = w1_ref[...]
        aux_ref[1:2, :] = jnp.broadcast_to(b1_ref[...], (1, D))


def _project(x3, w0, b0row, w1row, b1row):
    nbpg = 4
    rows = S // nbpg
    nblk = B * nbpg
    return pl.pallas_call(
        _proj_body,
        grid=(nblk,),
        in_specs=[
            pl.BlockSpec((1, rows, D), lambda i: (i // nbpg, i % nbpg, 0)),
            pl.BlockSpec((2 * D, D), lambda i: (0, 0)),
            pl.BlockSpec((1, D), lambda i: (0, 0)),
            pl.BlockSpec((1, D), lambda i: (0, 0)),
            pl.BlockSpec((1, 1), lambda i: (0, 0)),
        ],
        out_specs=[
            pl.BlockSpec((rows, D), lambda i: (i, 0)),
            pl.BlockSpec((rows, D), lambda i: (i, 0)),
            pl.BlockSpec((2, D), lambda i: (0, 0)),
        ],
        out_shape=[
            jax.ShapeDtypeStruct((B * S, D), jnp.float32),
            jax.ShapeDtypeStruct((B * S, D), jnp.float32),
            jax.ShapeDtypeStruct((2, D), jnp.float32),
        ],
    )(x3, w0, b0row, w1row, b1row)


def _unpack16(v):
    u = lax.bitcast_convert_type(v, jnp.uint32)
    lo = lax.bitcast_convert_type(lax.shift_left(u, jnp.uint32(16)), jnp.float32)
    hi = lax.bitcast_convert_type(u & jnp.uint32(0xFFFF0000), jnp.float32)
    return lo, hi


def _sc_body(a_hbm, c_hbm, dons_hbm, accs_hbm, aux_hbm,
             probs_out, sites_out,
             don_v, acc_v, gd_v, ga_v, a_rows, c_rows, probs_v,
             site_v, aux_v, idx2_v, shared_v,
             sa0, sa1, sc0, sc1):
    c = lax.axis_index("c")
    s = lax.axis_index("s")
    g_local = s // TPG
    g = c * GPC + g_local
    jbase = (s % TPG) * JPT
    flat = g * J + jbase

    pltpu.sync_copy(dons_hbm.at[g].at[pl.ds(jbase, JPT)], don_v)
    pltpu.sync_copy(accs_hbm.at[g].at[pl.ds(jbase, JPT)], acc_v)
    pltpu.sync_copy(aux_hbm, aux_v)

    zero16 = jnp.zeros((LANES,), jnp.float32)

    def zbody(i, _):
        site_v[i // 8, pl.ds((i % 8) * LANES, LANES)] = zero16
        return 0

    lax.fori_loop(0, S // LANES, zbody, 0)

    @pl.when(s == 0)
    def _():
        pltpu.sync_copy(site_v, shared_v.at[pl.ds(0, NS)])
        pltpu.sync_copy(site_v, shared_v.at[pl.ds(NS, NS)])

    goff = g * S
    for i in range(JPT // LANES):
        dvec = don_v[pl.ds(i * LANES, LANES)] + goff
        avec = acc_v[pl.ds(i * LANES, LANES)] + goff
        gd_v[i // (CHUNK // LANES), pl.ds((i % (CHUNK // LANES)) * LANES, LANES)] = dvec
        ga_v[i // (CHUNK // LANES), pl.ds((i % (CHUNK // LANES)) * LANES, LANES)] = avec

    plsc.subcore_barrier()

    ngrp = CHUNK // LANES
    jvecs = [gi * LANES + lax.iota(jnp.int32, LANES) for gi in range(ngrp)]
    sems_a = [sa0, sa1]
    sems_c = [sc0, sc1]

    def start(ch):
        buf = ch % 2
        cpa = pltpu.async_copy(a_hbm.at[gd_v.at[ch]], a_rows.at[buf], sems_a[buf])
        cpc = pltpu.async_copy(c_hbm.at[ga_v.at[ch]], c_rows.at[buf], sems_c[buf])
        return cpa, cpc

    def compute(ch, cpa, cpc):
        buf = ch % 2
        cpa.wait()
        cpc.wait()
        ar = a_rows.at[buf]
        cr = c_rows.at[buf]

        def k_body(kk, accs):
            kvd = (jnp.zeros((LANES,), jnp.int32) + kk
                   + lax.iota(jnp.int32, LANES)) & (D - 1)
            w = plsc.load_gather(aux_v, [kvd - kvd, kvd])
            out = []
            for gi in range(ngrp):
                va = plsc.load_gather(ar, [jvecs[gi], kvd])
                vc = plsc.load_gather(cr, [jvecs[gi], kvd])
                out.append(accs[gi] + jnp.maximum(va + vc, 0.0) * w)
            return tuple(out)

        accs = lax.fori_loop(
            0, D, k_body,
            tuple(jnp.zeros((LANES,), jnp.float32) for _ in range(ngrp)))
        for gi in range(ngrp):
            probs_v[pl.ds(ch * CHUNK + gi * LANES, LANES)] = accs[gi]

    pend = start(0)
    for ch in range(NCHUNK):
        nxt = start(ch + 1) if ch + 1 < NCHUNK else None
        compute(ch, *pend)
        pend = nxt

    ones = jnp.zeros((LANES,), jnp.int32) + 1
    b1v = plsc.load_gather(aux_v, [ones, ones - 1])

    def sig_body(i, _):
        x = probs_v[pl.ds(i * LANES, LANES)] + b1v
        probs_v[pl.ds(i * LANES, LANES)] = 1.0 / (1.0 + jnp.exp(-x))
        return 0

    lax.fori_loop(0, JPT // LANES, sig_body, 0)
    pltpu.sync_copy(probs_v, probs_out.at[g].at[pl.ds(jbase, JPT)])

    def scat_body(i, _):
        pv = probs_v[pl.ds(i * LANES, LANES)]
        dvec = don_v[pl.ds(i * LANES, LANES)]
        avec = acc_v[pl.ds(i * LANES, LANES)]
        plsc.addupdate_scatter(
            site_v, [lax.shift_right_logical(dvec, 7), dvec & 127], pv)
        plsc.addupdate_scatter(
            site_v, [lax.shift_right_logical(avec, 7), avec & 127], pv)
        return 0

    lax.fori_loop(0, JPT // LANES, scat_body, 0)

    idx2_v[0, pl.ds(0, LANES)] = g_local * NS + lax.iota(jnp.int32, LANES)
    pltpu.sync_copy(site_v, shared_v.at[idx2_v.at[0]], add=True)
    plsc.subcore_barrier()

    pltpu.sync_copy(shared_v.at[s],
                    sites_out.at[c * GPC].at[pl.ds(s * D, D)])
    pltpu.sync_copy(shared_v.at[s + NS],
                    sites_out.at[c * GPC + 1].at[pl.ds(s * D, D)])


def _sc_junctions(a_tab, c_tab, dons_flat, accs_flat, aux):
    mesh = plsc.VectorSubcoreMesh(core_axis_name="c", subcore_axis_name="s")
    f = functools.partial(
        pl.kernel,
        out_type=[
            jax.ShapeDtypeStruct((B, J), jnp.float32),
            jax.ShapeDtypeStruct((B, S), jnp.float32),
        ],
        mesh=mesh,
        compiler_params=pltpu.CompilerParams(needs_layout_passes=False, use_tc_tiling_on_sc=False),
        scratch_types=[
            pltpu.VMEM((JPT,), jnp.int32),
            pltpu.VMEM((JPT,), jnp.int32),
            pltpu.VMEM((NCHUNK, CHUNK), jnp.int32),
            pltpu.VMEM((NCHUNK, CHUNK), jnp.int32),
            pltpu.VMEM((2, CHUNK, D), jnp.float32),
            pltpu.VMEM((2, CHUNK, D), jnp.float32),
            pltpu.VMEM((JPT,), jnp.float32),
            pltpu.VMEM((NS, D), jnp.float32),
            pltpu.VMEM((2, D), jnp.float32),
            pltpu.VMEM((1, LANES), jnp.int32),
            pltpu.VMEM_SHARED((GPC * NS, D), jnp.float32),
            pltpu.SemaphoreType.DMA,
            pltpu.SemaphoreType.DMA,
            pltpu.SemaphoreType.DMA,
            pltpu.SemaphoreType.DMA,
        ],
    )(_sc_body)
    return f(a_tab, c_tab, dons_flat, accs_flat, aux)


def kernel(splice_site_reps, gene_start_rep, gene_end_rep, W0, b0, W1, b1,
           junction_dons, junction_accs):
    b0row = b0.reshape(1, D)
    w1row = W1.reshape(1, D)
    b1row = b1.reshape(1, 1)
    a_tab, c_tab, aux = _project(splice_site_reps, W0, b0row, w1row, b1row)

    dons = junction_dons.astype(jnp.int32)
    accs = junction_accs.astype(jnp.int32)

    probs, sites = _sc_junctions(a_tab, c_tab, dons, accs, aux)
    return probs, sites


# speedup vs baseline: 1.1402x; 1.0839x over previous
import functools

import jax
import jax.numpy as jnp
from jax import lax
from jax.experimental import pallas as pl
from jax.experimental.pallas import tpu as pltpu
from jax.experimental.pallas import tpu_sc as plsc

B = 4
S = 2048
J = 4096
D = 128
DP = D // 2

NC = 2
NS = 16
GPC = B // NC
TPG = NS // GPC
JPT = J // TPG
CHUNK = 128
NCHUNK = JPT // CHUNK
LANES = 16


def _pack_halves(x):
    lo = x[..., :DP].astype(jnp.bfloat16).astype(jnp.float32)
    hi = x[..., DP:].astype(jnp.bfloat16).astype(jnp.float32)
    lo_u = lax.shift_right_logical(lax.bitcast_convert_type(lo, jnp.uint32), jnp.uint32(16))
    hi_u = lax.bitcast_convert_type(hi, jnp.uint32) & jnp.uint32(0xFFFF0000)
    return lax.bitcast_convert_type(lo_u | hi_u, jnp.float32)


def _proj_body(x_ref, w0_ref, b0_ref, w1_ref, b1_ref, a_ref, c_ref, aux_ref):
    x = x_ref[0]
    wd = w0_ref[:D, :]
    wa = w0_ref[D:, :]
    a = jnp.dot(x, wd, preferred_element_type=jnp.float32) + b0_ref[...]
    c = jnp.dot(x, wa, preferred_element_type=jnp.float32)
    a_ref[...] = a
    c_ref[...] = c

    @pl.when(pl.program_id(0) == 0)
    def _():
        aux_ref[0:1, :] = w1_ref[...]
        aux_ref[1:2, :] = jnp.broadcast_to(b1_ref[...], (1, D))


def _project(x3, w0, b0row, w1row, b1row):
    nbpg = 2
    rows = S // nbpg
    nblk = B * nbpg
    return pl.pallas_call(
        _proj_body,
        grid=(nblk,),
        in_specs=[
            pl.BlockSpec((1, rows, D), lambda i: (i // nbpg, i % nbpg, 0)),
            pl.BlockSpec((2 * D, D), lambda i: (0, 0)),
            pl.BlockSpec((1, D), lambda i: (0, 0)),
            pl.BlockSpec((1, D), lambda i: (0, 0)),
            pl.BlockSpec((1, 1), lambda i: (0, 0)),
        ],
        out_specs=[
            pl.BlockSpec((rows, D), lambda i: (i, 0)),
            pl.BlockSpec((rows, D), lambda i: (i, 0)),
            pl.BlockSpec((2, D), lambda i: (0, 0)),
        ],
        out_shape=[
            jax.ShapeDtypeStruct((B * S, D), jnp.float32),
            jax.ShapeDtypeStruct((B * S, D), jnp.float32),
            jax.ShapeDtypeStruct((2, D), jnp.float32),
        ],
    )(x3, w0, b0row, w1row, b1row)


def _unpack16(v):
    u = lax.bitcast_convert_type(v, jnp.uint32)
    lo = lax.bitcast_convert_type(lax.shift_left(u, jnp.uint32(16)), jnp.float32)
    hi = lax.bitcast_convert_type(u & jnp.uint32(0xFFFF0000), jnp.float32)
    return lo, hi


def _sc_body(a_hbm, c_hbm, dons_hbm, accs_hbm, aux_hbm,
             probs_out, sites_out,
             don_v, acc_v, gd_v, ga_v, a_rows, c_rows, probs_v,
             site_v, aux_v, idx2_v, shared_v,
             sa0, sa1, sc0, sc1):
    c = lax.axis_index("c")
    s = lax.axis_index("s")
    g_local = s // TPG
    g = c * GPC + g_local
    jbase = (s % TPG) * JPT
    flat = g * J + jbase

    pltpu.sync_copy(dons_hbm.at[g].at[pl.ds(jbase, JPT)], don_v)
    pltpu.sync_copy(accs_hbm.at[g].at[pl.ds(jbase, JPT)], acc_v)
    pltpu.sync_copy(aux_hbm, aux_v)

    zero16 = jnp.zeros((LANES,), jnp.float32)

    def zbody(i, _):
        site_v[i // 8, pl.ds((i % 8) * LANES, LANES)] = zero16
        return 0

    lax.fori_loop(0, S // LANES, zbody, 0)

    @pl.when(s == 0)
    def _():
        pltpu.sync_copy(site_v, shared_v.at[pl.ds(0, NS)])
        pltpu.sync_copy(site_v, shared_v.at[pl.ds(NS, NS)])

    goff = g * S
    for i in range(JPT // LANES):
        dvec = don_v[pl.ds(i * LANES, LANES)] + goff
        avec = acc_v[pl.ds(i * LANES, LANES)] + goff
        gd_v[i // (CHUNK // LANES), pl.ds((i % (CHUNK // LANES)) * LANES, LANES)] = dvec
        ga_v[i // (CHUNK // LANES), pl.ds((i % (CHUNK // LANES)) * LANES, LANES)] = avec

    plsc.subcore_barrier()

    ngrp = CHUNK // LANES
    jvecs = [gi * LANES + lax.iota(jnp.int32, LANES) for gi in range(ngrp)]
    sems_a = [sa0, sa1]
    sems_c = [sc0, sc1]

    def start(ch):
        buf = ch % 2
        cpa = pltpu.async_copy(a_hbm.at[gd_v.at[ch]], a_rows.at[buf], sems_a[buf])
        cpc = pltpu.async_copy(c_hbm.at[ga_v.at[ch]], c_rows.at[buf], sems_c[buf])
        return cpa, cpc

    def compute(ch, cpa, cpc):
        buf = ch % 2
        cpa.wait()
        cpc.wait()
        ar = a_rows.at[buf]
        cr = c_rows.at[buf]

        def k_body(kk, accs):
            kvd = (jnp.zeros((LANES,), jnp.int32) + kk
                   + lax.iota(jnp.int32, LANES)) & (D - 1)
            w = plsc.load_gather(aux_v, [kvd - kvd, kvd])
            out = []
            for gi in range(ngrp):
                va = plsc.load_gather(ar, [jvecs[gi], kvd])
                vc = plsc.load_gather(cr, [jvecs[gi], kvd])
                out.append(accs[gi] + jnp.maximum(va + vc, 0.0) * w)
            return tuple(out)

        accs = lax.fori_loop(
            0, D, k_body,
            tuple(jnp.zeros((LANES,), jnp.float32) for _ in range(ngrp)))
        for gi in range(ngrp):
            probs_v[pl.ds(ch * CHUNK + gi * LANES, LANES)] = accs[gi]

    pend = start(0)
    for ch in range(NCHUNK):
        nxt = start(ch + 1) if ch + 1 < NCHUNK else None
        compute(ch, *pend)
        pend = nxt

    ones = jnp.zeros((LANES,), jnp.int32) + 1
    b1v = plsc.load_gather(aux_v, [ones, ones - 1])

    def sig_body(i, _):
        x = probs_v[pl.ds(i * LANES, LANES)] + b1v
        probs_v[pl.ds(i * LANES, LANES)] = 1.0 / (1.0 + jnp.exp(-x))
        return 0

    lax.fori_loop(0, JPT // LANES, sig_body, 0)
    pltpu.sync_copy(probs_v, probs_out.at[g].at[pl.ds(jbase, JPT)])

    def scat_body(i, _):
        pv = probs_v[pl.ds(i * LANES, LANES)]
        dvec = don_v[pl.ds(i * LANES, LANES)]
        avec = acc_v[pl.ds(i * LANES, LANES)]
        plsc.addupdate_scatter(
            site_v, [lax.shift_right_logical(dvec, 7), dvec & 127], pv)
        plsc.addupdate_scatter(
            site_v, [lax.shift_right_logical(avec, 7), avec & 127], pv)
        return 0

    lax.fori_loop(0, JPT // LANES, scat_body, 0)

    idx2_v[0, pl.ds(0, LANES)] = g_local * NS + lax.iota(jnp.int32, LANES)
    pltpu.sync_copy(site_v, shared_v.at[idx2_v.at[0]], add=True)
    plsc.subcore_barrier()

    pltpu.sync_copy(shared_v.at[s],
                    sites_out.at[c * GPC].at[pl.ds(s * D, D)])
    pltpu.sync_copy(shared_v.at[s + NS],
                    sites_out.at[c * GPC + 1].at[pl.ds(s * D, D)])


def _sc_junctions(a_tab, c_tab, dons_flat, accs_flat, aux):
    mesh = plsc.VectorSubcoreMesh(core_axis_name="c", subcore_axis_name="s")
    f = functools.partial(
        pl.kernel,
        out_type=[
            jax.ShapeDtypeStruct((B, J), jnp.float32),
            jax.ShapeDtypeStruct((B, S), jnp.float32),
        ],
        mesh=mesh,
        compiler_params=pltpu.CompilerParams(needs_layout_passes=False, use_tc_tiling_on_sc=False),
        scratch_types=[
            pltpu.VMEM((JPT,), jnp.int32),
            pltpu.VMEM((JPT,), jnp.int32),
            pltpu.VMEM((NCHUNK, CHUNK), jnp.int32),
            pltpu.VMEM((NCHUNK, CHUNK), jnp.int32),
            pltpu.VMEM((2, CHUNK, D), jnp.float32),
            pltpu.VMEM((2, CHUNK, D), jnp.float32),
            pltpu.VMEM((JPT,), jnp.float32),
            pltpu.VMEM((NS, D), jnp.float32),
            pltpu.VMEM((2, D), jnp.float32),
            pltpu.VMEM((1, LANES), jnp.int32),
            pltpu.VMEM_SHARED((GPC * NS, D), jnp.float32),
            pltpu.SemaphoreType.DMA,
            pltpu.SemaphoreType.DMA,
            pltpu.SemaphoreType.DMA,
            pltpu.SemaphoreType.DMA,
        ],
    )(_sc_body)
    return f(a_tab, c_tab, dons_flat, accs_flat, aux)


def kernel(splice_site_reps, gene_start_rep, gene_end_rep, W0, b0, W1, b1,
           junction_dons, junction_accs):
    b0row = b0.reshape(1, D)
    w1row = W1.reshape(1, D)
    b1row = b1.reshape(1, 1)
    a_tab, c_tab, aux = _project(splice_site_reps, W0, b0row, w1row, b1row)

    dons = junction_dons.astype(jnp.int32)
    accs = junction_accs.astype(jnp.int32)

    probs, sites = _sc_junctions(a_tab, c_tab, dons, accs, aux)
    return probs, sites


# speedup vs baseline: 1.1966x; 1.0494x over previous
import functools

import jax
import jax.numpy as jnp
from jax import lax
from jax.experimental import pallas as pl
from jax.experimental.pallas import tpu as pltpu
from jax.experimental.pallas import tpu_sc as plsc

B = 4
S = 2048
J = 4096
D = 128
DP = D // 2

NC = 2
NS = 16
GPC = B // NC
TPG = NS // GPC
JPT = J // TPG
CHUNK = 128
NCHUNK = JPT // CHUNK
LANES = 16


def _pack_halves(x):
    lo = x[..., :DP].astype(jnp.bfloat16).astype(jnp.float32)
    hi = x[..., DP:].astype(jnp.bfloat16).astype(jnp.float32)
    lo_u = lax.shift_right_logical(lax.bitcast_convert_type(lo, jnp.uint32), jnp.uint32(16))
    hi_u = lax.bitcast_convert_type(hi, jnp.uint32) & jnp.uint32(0xFFFF0000)
    return lax.bitcast_convert_type(lo_u | hi_u, jnp.float32)


def _proj_body(x_ref, w0_ref, b0_ref, w1_ref, b1_ref, a_ref, c_ref, aux_ref):
    x = x_ref[0]
    wd = w0_ref[:D, :]
    wa = w0_ref[D:, :]
    a = jnp.dot(x, wd, preferred_element_type=jnp.float32) + b0_ref[...]
    c = jnp.dot(x, wa, preferred_element_type=jnp.float32)
    a_ref[...] = a
    c_ref[...] = c

    @pl.when(pl.program_id(0) == 0)
    def _():
        aux_ref[0:1, :] = w1_ref[...]
        aux_ref[1:2, :] = jnp.broadcast_to(b1_ref[...], (1, D))


def _project(x3, w0, b0row, w1row, b1row):
    nbpg = 1
    rows = S // nbpg
    nblk = B * nbpg
    return pl.pallas_call(
        _proj_body,
        grid=(nblk,),
        in_specs=[
            pl.BlockSpec((1, rows, D), lambda i: (i // nbpg, i % nbpg, 0)),
            pl.BlockSpec((2 * D, D), lambda i: (0, 0)),
            pl.BlockSpec((1, D), lambda i: (0, 0)),
            pl.BlockSpec((1, D), lambda i: (0, 0)),
            pl.BlockSpec((1, 1), lambda i: (0, 0)),
        ],
        out_specs=[
            pl.BlockSpec((rows, D), lambda i: (i, 0)),
            pl.BlockSpec((rows, D), lambda i: (i, 0)),
            pl.BlockSpec((2, D), lambda i: (0, 0)),
        ],
        out_shape=[
            jax.ShapeDtypeStruct((B * S, D), jnp.float32),
            jax.ShapeDtypeStruct((B * S, D), jnp.float32),
            jax.ShapeDtypeStruct((2, D), jnp.float32),
        ],
    )(x3, w0, b0row, w1row, b1row)


def _unpack16(v):
    u = lax.bitcast_convert_type(v, jnp.uint32)
    lo = lax.bitcast_convert_type(lax.shift_left(u, jnp.uint32(16)), jnp.float32)
    hi = lax.bitcast_convert_type(u & jnp.uint32(0xFFFF0000), jnp.float32)
    return lo, hi


def _sc_body(a_hbm, c_hbm, dons_hbm, accs_hbm, aux_hbm,
             probs_out, sites_out,
             don_v, acc_v, gd_v, ga_v, a_rows, c_rows, probs_v,
             site_v, aux_v, idx2_v, shared_v,
             sa0, sa1, sc0, sc1):
    c = lax.axis_index("c")
    s = lax.axis_index("s")
    g_local = s // TPG
    g = c * GPC + g_local
    jbase = (s % TPG) * JPT
    flat = g * J + jbase

    pltpu.sync_copy(dons_hbm.at[g].at[pl.ds(jbase, JPT)], don_v)
    pltpu.sync_copy(accs_hbm.at[g].at[pl.ds(jbase, JPT)], acc_v)
    pltpu.sync_copy(aux_hbm, aux_v)

    zero16 = jnp.zeros((LANES,), jnp.float32)

    def zbody(i, _):
        site_v[i // 8, pl.ds((i % 8) * LANES, LANES)] = zero16
        return 0

    lax.fori_loop(0, S // LANES, zbody, 0)

    @pl.when(s == 0)
    def _():
        pltpu.sync_copy(site_v, shared_v.at[pl.ds(0, NS)])
        pltpu.sync_copy(site_v, shared_v.at[pl.ds(NS, NS)])

    goff = g * S
    for i in range(JPT // LANES):
        dvec = don_v[pl.ds(i * LANES, LANES)] + goff
        avec = acc_v[pl.ds(i * LANES, LANES)] + goff
        gd_v[i // (CHUNK // LANES), pl.ds((i % (CHUNK // LANES)) * LANES, LANES)] = dvec
        ga_v[i // (CHUNK // LANES), pl.ds((i % (CHUNK // LANES)) * LANES, LANES)] = avec

    plsc.subcore_barrier()

    ngrp = CHUNK // LANES
    jvecs = [gi * LANES + lax.iota(jnp.int32, LANES) for gi in range(ngrp)]
    sems_a = [sa0, sa1]
    sems_c = [sc0, sc1]

    def start(ch):
        buf = ch % 2
        cpa = pltpu.async_copy(a_hbm.at[gd_v.at[ch]], a_rows.at[buf], sems_a[buf])
        cpc = pltpu.async_copy(c_hbm.at[ga_v.at[ch]], c_rows.at[buf], sems_c[buf])
        return cpa, cpc

    def compute(ch, cpa, cpc):
        buf = ch % 2
        cpa.wait()
        cpc.wait()
        ar = a_rows.at[buf]
        cr = c_rows.at[buf]

        def k_body(kk, accs):
            kvd = (jnp.zeros((LANES,), jnp.int32) + kk
                   + lax.iota(jnp.int32, LANES)) & (D - 1)
            w = plsc.load_gather(aux_v, [kvd - kvd, kvd])
            out = []
            for gi in range(ngrp):
                va = plsc.load_gather(ar, [jvecs[gi], kvd])
                vc = plsc.load_gather(cr, [jvecs[gi], kvd])
                out.append(accs[gi] + jnp.maximum(va + vc, 0.0) * w)
            return tuple(out)

        accs = lax.fori_loop(
            0, D, k_body,
            tuple(jnp.zeros((LANES,), jnp.float32) for _ in range(ngrp)))
        for gi in range(ngrp):
            probs_v[pl.ds(ch * CHUNK + gi * LANES, LANES)] = accs[gi]

    pend = start(0)
    for ch in range(NCHUNK):
        nxt = start(ch + 1) if ch + 1 < NCHUNK else None
        compute(ch, *pend)
        pend = nxt

    ones = jnp.zeros((LANES,), jnp.int32) + 1
    b1v = plsc.load_gather(aux_v, [ones, ones - 1])

    def sig_body(i, _):
        x = probs_v[pl.ds(i * LANES, LANES)] + b1v
        probs_v[pl.ds(i * LANES, LANES)] = 1.0 / (1.0 + jnp.exp(-x))
        return 0

    lax.fori_loop(0, JPT // LANES, sig_body, 0)
    pltpu.sync_copy(probs_v, probs_out.at[g].at[pl.ds(jbase, JPT)])

    def scat_body(i, _):
        pv = probs_v[pl.ds(i * LANES, LANES)]
        dvec = don_v[pl.ds(i * LANES, LANES)]
        avec = acc_v[pl.ds(i * LANES, LANES)]
        plsc.addupdate_scatter(
            site_v, [lax.shift_right_logical(dvec, 7), dvec & 127], pv)
        plsc.addupdate_scatter(
            site_v, [lax.shift_right_logical(avec, 7), avec & 127], pv)
        return 0

    lax.fori_loop(0, JPT // LANES, scat_body, 0)

    idx2_v[0, pl.ds(0, LANES)] = g_local * NS + lax.iota(jnp.int32, LANES)
    pltpu.sync_copy(site_v, shared_v.at[idx2_v.at[0]], add=True)
    plsc.subcore_barrier()

    pltpu.sync_copy(shared_v.at[s],
                    sites_out.at[c * GPC].at[pl.ds(s * D, D)])
    pltpu.sync_copy(shared_v.at[s + NS],
                    sites_out.at[c * GPC + 1].at[pl.ds(s * D, D)])


def _sc_junctions(a_tab, c_tab, dons_flat, accs_flat, aux):
    mesh = plsc.VectorSubcoreMesh(core_axis_name="c", subcore_axis_name="s")
    f = functools.partial(
        pl.kernel,
        out_type=[
            jax.ShapeDtypeStruct((B, J), jnp.float32),
            jax.ShapeDtypeStruct((B, S), jnp.float32),
        ],
        mesh=mesh,
        compiler_params=pltpu.CompilerParams(needs_layout_passes=False, use_tc_tiling_on_sc=False),
        scratch_types=[
            pltpu.VMEM((JPT,), jnp.int32),
            pltpu.VMEM((JPT,), jnp.int32),
            pltpu.VMEM((NCHUNK, CHUNK), jnp.int32),
            pltpu.VMEM((NCHUNK, CHUNK), jnp.int32),
            pltpu.VMEM((2, CHUNK, D), jnp.float32),
            pltpu.VMEM((2, CHUNK, D), jnp.float32),
            pltpu.VMEM((JPT,), jnp.float32),
            pltpu.VMEM((NS, D), jnp.float32),
            pltpu.VMEM((2, D), jnp.float32),
            pltpu.VMEM((1, LANES), jnp.int32),
            pltpu.VMEM_SHARED((GPC * NS, D), jnp.float32),
            pltpu.SemaphoreType.DMA,
            pltpu.SemaphoreType.DMA,
            pltpu.SemaphoreType.DMA,
            pltpu.SemaphoreType.DMA,
        ],
    )(_sc_body)
    return f(a_tab, c_tab, dons_flat, accs_flat, aux)


def kernel(splice_site_reps, gene_start_rep, gene_end_rep, W0, b0, W1, b1,
           junction_dons, junction_accs):
    b0row = b0.reshape(1, D)
    w1row = W1.reshape(1, D)
    b1row = b1.reshape(1, 1)
    a_tab, c_tab, aux = _project(splice_site_reps, W0, b0row, w1row, b1row)

    dons = junction_dons.astype(jnp.int32)
    accs = junction_accs.astype(jnp.int32)

    probs, sites = _sc_junctions(a_tab, c_tab, dons, accs, aux)
    return probs, sites


# speedup vs baseline: 1.2166x; 1.0167x over previous
import functools

import jax
import jax.numpy as jnp
from jax import lax
from jax.experimental import pallas as pl
from jax.experimental.pallas import tpu as pltpu
from jax.experimental.pallas import tpu_sc as plsc

B = 4
S = 2048
J = 4096
D = 128
DP = D // 2

NC = 2
NS = 16
GPC = B // NC
TPG = NS // GPC
JPT = J // TPG
CHUNK = 128
NCHUNK = JPT // CHUNK
LANES = 16


def _pack_halves(x):
    lo = x[..., :DP].astype(jnp.bfloat16).astype(jnp.float32)
    hi = x[..., DP:].astype(jnp.bfloat16).astype(jnp.float32)
    lo_u = lax.shift_right_logical(lax.bitcast_convert_type(lo, jnp.uint32), jnp.uint32(16))
    hi_u = lax.bitcast_convert_type(hi, jnp.uint32) & jnp.uint32(0xFFFF0000)
    return lax.bitcast_convert_type(lo_u | hi_u, jnp.float32)


def _proj_body(x_ref, w0_ref, b0_ref, w1_ref, b1_ref, a_ref, c_ref, aux_ref):
    x = x_ref[...].reshape(2 * S, D)
    wd = w0_ref[:D, :]
    wa = w0_ref[D:, :]
    a = jnp.dot(x, wd, preferred_element_type=jnp.float32) + b0_ref[...]
    c = jnp.dot(x, wa, preferred_element_type=jnp.float32)
    a_ref[...] = a
    c_ref[...] = c

    @pl.when(pl.program_id(0) == 0)
    def _():
        aux_ref[0:1, :] = w1_ref[...]
        aux_ref[1:2, :] = jnp.broadcast_to(b1_ref[...], (1, D))


def _project(x3, w0, b0row, w1row, b1row):
    rows = 2 * S
    nblk = 2
    return pl.pallas_call(
        _proj_body,
        grid=(nblk,),
        in_specs=[
            pl.BlockSpec((2, S, D), lambda i: (i, 0, 0)),
            pl.BlockSpec((2 * D, D), lambda i: (0, 0)),
            pl.BlockSpec((1, D), lambda i: (0, 0)),
            pl.BlockSpec((1, D), lambda i: (0, 0)),
            pl.BlockSpec((1, 1), lambda i: (0, 0)),
        ],
        out_specs=[
            pl.BlockSpec((rows, D), lambda i: (i, 0)),
            pl.BlockSpec((rows, D), lambda i: (i, 0)),
            pl.BlockSpec((2, D), lambda i: (0, 0)),
        ],
        out_shape=[
            jax.ShapeDtypeStruct((B * S, D), jnp.float32),
            jax.ShapeDtypeStruct((B * S, D), jnp.float32),
            jax.ShapeDtypeStruct((2, D), jnp.float32),
        ],
    )(x3, w0, b0row, w1row, b1row)


def _unpack16(v):
    u = lax.bitcast_convert_type(v, jnp.uint32)
    lo = lax.bitcast_convert_type(lax.shift_left(u, jnp.uint32(16)), jnp.float32)
    hi = lax.bitcast_convert_type(u & jnp.uint32(0xFFFF0000), jnp.float32)
    return lo, hi


def _sc_body(a_hbm, c_hbm, dons_hbm, accs_hbm, aux_hbm,
             probs_out, sites_out,
             don_v, acc_v, gd_v, ga_v, a_rows, c_rows, probs_v,
             site_v, aux_v, idx2_v, shared_v,
             sa0, sa1, sc0, sc1):
    c = lax.axis_index("c")
    s = lax.axis_index("s")
    g_local = s // TPG
    g = c * GPC + g_local
    jbase = (s % TPG) * JPT
    flat = g * J + jbase

    pltpu.sync_copy(dons_hbm.at[g].at[pl.ds(jbase, JPT)], don_v)
    pltpu.sync_copy(accs_hbm.at[g].at[pl.ds(jbase, JPT)], acc_v)
    pltpu.sync_copy(aux_hbm, aux_v)

    zero16 = jnp.zeros((LANES,), jnp.float32)

    def zbody(i, _):
        site_v[i // 8, pl.ds((i % 8) * LANES, LANES)] = zero16
        return 0

    lax.fori_loop(0, S // LANES, zbody, 0)

    @pl.when(s == 0)
    def _():
        pltpu.sync_copy(site_v, shared_v.at[pl.ds(0, NS)])
        pltpu.sync_copy(site_v, shared_v.at[pl.ds(NS, NS)])

    goff = g * S
    for i in range(JPT // LANES):
        dvec = don_v[pl.ds(i * LANES, LANES)] + goff
        avec = acc_v[pl.ds(i * LANES, LANES)] + goff
        gd_v[i // (CHUNK // LANES), pl.ds((i % (CHUNK // LANES)) * LANES, LANES)] = dvec
        ga_v[i // (CHUNK // LANES), pl.ds((i % (CHUNK // LANES)) * LANES, LANES)] = avec

    plsc.subcore_barrier()

    ngrp = CHUNK // LANES
    jvecs = [gi * LANES + lax.iota(jnp.int32, LANES) for gi in range(ngrp)]
    sems_a = [sa0, sa1]
    sems_c = [sc0, sc1]

    def start(ch):
        buf = ch % 2
        cpa = pltpu.async_copy(a_hbm.at[gd_v.at[ch]], a_rows.at[buf], sems_a[buf])
        cpc = pltpu.async_copy(c_hbm.at[ga_v.at[ch]], c_rows.at[buf], sems_c[buf])
        return cpa, cpc

    def compute(ch, cpa, cpc):
        buf = ch % 2
        cpa.wait()
        cpc.wait()
        ar = a_rows.at[buf]
        cr = c_rows.at[buf]

        def k_body(kk, accs):
            kvd = (jnp.zeros((LANES,), jnp.int32) + kk
                   + lax.iota(jnp.int32, LANES)) & (D - 1)
            w = plsc.load_gather(aux_v, [kvd - kvd, kvd])
            out = []
            for gi in range(ngrp):
                va = plsc.load_gather(ar, [jvecs[gi], kvd])
                vc = plsc.load_gather(cr, [jvecs[gi], kvd])
                out.append(accs[gi] + jnp.maximum(va + vc, 0.0) * w)
            return tuple(out)

        accs = lax.fori_loop(
            0, D, k_body,
            tuple(jnp.zeros((LANES,), jnp.float32) for _ in range(ngrp)))
        for gi in range(ngrp):
            probs_v[pl.ds(ch * CHUNK + gi * LANES, LANES)] = accs[gi]

    pend = start(0)
    for ch in range(NCHUNK):
        nxt = start(ch + 1) if ch + 1 < NCHUNK else None
        compute(ch, *pend)
        pend = nxt

    ones = jnp.zeros((LANES,), jnp.int32) + 1
    b1v = plsc.load_gather(aux_v, [ones, ones - 1])

    def sig_body(i, _):
        x = probs_v[pl.ds(i * LANES, LANES)] + b1v
        probs_v[pl.ds(i * LANES, LANES)] = 1.0 / (1.0 + jnp.exp(-x))
        return 0

    lax.fori_loop(0, JPT // LANES, sig_body, 0)
    pltpu.sync_copy(probs_v, probs_out.at[g].at[pl.ds(jbase, JPT)])

    def scat_body(i, _):
        pv = probs_v[pl.ds(i * LANES, LANES)]
        dvec = don_v[pl.ds(i * LANES, LANES)]
        avec = acc_v[pl.ds(i * LANES, LANES)]
        plsc.addupdate_scatter(
            site_v, [lax.shift_right_logical(dvec, 7), dvec & 127], pv)
        plsc.addupdate_scatter(
            site_v, [lax.shift_right_logical(avec, 7), avec & 127], pv)
        return 0

    lax.fori_loop(0, JPT // LANES, scat_body, 0)

    idx2_v[0, pl.ds(0, LANES)] = g_local * NS + lax.iota(jnp.int32, LANES)
    pltpu.sync_copy(site_v, shared_v.at[idx2_v.at[0]], add=True)
    plsc.subcore_barrier()

    pltpu.sync_copy(shared_v.at[s],
                    sites_out.at[c * GPC].at[pl.ds(s * D, D)])
    pltpu.sync_copy(shared_v.at[s + NS],
                    sites_out.at[c * GPC + 1].at[pl.ds(s * D, D)])


def _sc_junctions(a_tab, c_tab, dons_flat, accs_flat, aux):
    mesh = plsc.VectorSubcoreMesh(core_axis_name="c", subcore_axis_name="s")
    f = functools.partial(
        pl.kernel,
        out_type=[
            jax.ShapeDtypeStruct((B, J), jnp.float32),
            jax.ShapeDtypeStruct((B, S), jnp.float32),
        ],
        mesh=mesh,
        compiler_params=pltpu.CompilerParams(needs_layout_passes=False, use_tc_tiling_on_sc=False),
        scratch_types=[
            pltpu.VMEM((JPT,), jnp.int32),
            pltpu.VMEM((JPT,), jnp.int32),
            pltpu.VMEM((NCHUNK, CHUNK), jnp.int32),
            pltpu.VMEM((NCHUNK, CHUNK), jnp.int32),
            pltpu.VMEM((2, CHUNK, D), jnp.float32),
            pltpu.VMEM((2, CHUNK, D), jnp.float32),
            pltpu.VMEM((JPT,), jnp.float32),
            pltpu.VMEM((NS, D), jnp.float32),
            pltpu.VMEM((2, D), jnp.float32),
            pltpu.VMEM((1, LANES), jnp.int32),
            pltpu.VMEM_SHARED((GPC * NS, D), jnp.float32),
            pltpu.SemaphoreType.DMA,
            pltpu.SemaphoreType.DMA,
            pltpu.SemaphoreType.DMA,
            pltpu.SemaphoreType.DMA,
        ],
    )(_sc_body)
    return f(a_tab, c_tab, dons_flat, accs_flat, aux)


def kernel(splice_site_reps, gene_start_rep, gene_end_rep, W0, b0, W1, b1,
           junction_dons, junction_accs):
    b0row = b0.reshape(1, D)
    w1row = W1.reshape(1, D)
    b1row = b1.reshape(1, 1)
    a_tab, c_tab, aux = _project(splice_site_reps, W0, b0row, w1row, b1row)

    dons = junction_dons.astype(jnp.int32)
    accs = junction_accs.astype(jnp.int32)

    probs, sites = _sc_junctions(a_tab, c_tab, dons, accs, aux)
    return probs, sites
